# Initial kernel scaffold; baseline (speedup 1.0000x reference)
import functools

import jax
import jax.numpy as jnp
from jax import lax
from jax.experimental import pallas as pl
from jax.experimental.pallas import tpu as pltpu
from jax.experimental.pallas import tpu_sc as plsc

N = 10000
D = 128
K = 128
NC = 2
NS = 16
NW = NC * NS
ROWS_PER_TILE = N // NS
ZROWS = 125


def _matvec_body(x_ref, w_ref, o_ref):
    o_ref[...] = jax.lax.dot_general(
        w_ref[...], x_ref[...], (((1,), (1,)), ((), ())),
        preferred_element_type=jnp.float32)


def _combine_body(p_ref, o_ref):
    o_ref[...] = p_ref[0] + p_ref[1]


def _sc_body(ept, x_hbm, srcp, dstp, ewp, a2_hbm, out_hbm,
             h_sh, asrc_v, adst_v, src_v, dst_v, ew_v, coeff_v, xrows, zbuf,
             sem):
    c = lax.axis_index("c")
    s = lax.axis_index("s")
    wid = s * NC + c

    pltpu.sync_copy(a2_hbm.at[0], asrc_v)
    pltpu.sync_copy(a2_hbm.at[1], adst_v)

    @pl.loop(0, ZROWS * D // 16)
    def _zero(i):
        zbuf[pl.ds(i * 16, 16)] = jnp.zeros((16,), jnp.float32)

    zmat = zbuf.reshape(ZROWS, D)
    for z in range(ROWS_PER_TILE // ZROWS):
        pltpu.sync_copy(zmat, h_sh.at[pl.ds(s * ROWS_PER_TILE + z * ZROWS, ZROWS)])
    plsc.subcore_barrier()

    @pl.loop(0, ept // K)
    def _chunk(i):
        base = wid * ept + i * K
        pltpu.sync_copy(srcp.at[pl.ds(base, K)], src_v)
        pltpu.sync_copy(dstp.at[pl.ds(base, K)], dst_v)
        pltpu.sync_copy(ewp.at[pl.ds(base, K)], ew_v)

        gather = pltpu.async_copy(x_hbm.at[src_v], xrows, sem)

        for t in range(K // 16):
            sl = pl.ds(t * 16, 16)
            a_s = plsc.load_gather(asrc_v, [src_v[sl]])
            a_d = plsc.load_gather(adst_v, [dst_v[sl]])
            z = a_s + a_d
            th = 1.0 - 2.0 / (jnp.exp(z + z) + 1.0)
            coeff_v[sl] = th * ew_v[sl]

        gather.wait()

        @pl.loop(0, K)
        def _scale(j):
            cvec = plsc.load_gather(coeff_v, [jnp.full((16,), j, jnp.int32)])
            for t in range(D // 16):
                sl = pl.ds(t * 16, 16)
                xrows[j, sl] = xrows[j, sl] * cvec

        pltpu.sync_copy(xrows, h_sh.at[dst_v], add=True)

    plsc.subcore_barrier()

    pltpu.sync_copy(h_sh.at[pl.ds(s * ROWS_PER_TILE, ROWS_PER_TILE)],
                    out_hbm.at[c, pl.ds(s * ROWS_PER_TILE, ROWS_PER_TILE)])


def kernel(x, edge_index, ew, w_src, w_dst):
    E = ew.shape[0]
    src = edge_index[0].astype(jnp.int32)
    dst = edge_index[1].astype(jnp.int32)

    ept = -(-E // (NW * K)) * K
    e_pad = ept * NW
    pad = e_pad - E
    if pad:
        src = jnp.concatenate([src, jnp.zeros((pad,), jnp.int32)])
        dst = jnp.concatenate([dst, jnp.zeros((pad,), jnp.int32)])
        ew = jnp.concatenate([ew, jnp.zeros((pad,), jnp.float32)])

    wb = jnp.stack([w_src, w_dst])
    blk = 1000
    a2 = pl.pallas_call(
        _matvec_body,
        grid=(N // blk,),
        in_specs=[pl.BlockSpec((blk, D), lambda i: (i, 0)),
                  pl.BlockSpec((2, D), lambda i: (0, 0))],
        out_specs=pl.BlockSpec((2, blk), lambda i: (0, i)),
        out_shape=jax.ShapeDtypeStruct((2, N), jnp.float32),
    )(x, wb)

    sc = pl.kernel(
        functools.partial(_sc_body, ept),
        out_type=jax.ShapeDtypeStruct((NC, N, D), jnp.float32),
        mesh=plsc.VectorSubcoreMesh(core_axis_name="c", subcore_axis_name="s"),
        scratch_types=[
            pltpu.VMEM_SHARED((N, D), jnp.float32),
            pltpu.VMEM((N,), jnp.float32),
            pltpu.VMEM((N,), jnp.float32),
            pltpu.VMEM((K,), jnp.int32),
            pltpu.VMEM((K,), jnp.int32),
            pltpu.VMEM((K,), jnp.float32),
            pltpu.VMEM((K,), jnp.float32),
            pltpu.VMEM((K, D), jnp.float32),
            pltpu.VMEM((ZROWS * D,), jnp.float32),
            pltpu.SemaphoreType.DMA,
        ],
    )
    partials = sc(x, src, dst, ew, a2)

    h = pl.pallas_call(
        _combine_body,
        grid=(N // blk,),
        in_specs=[pl.BlockSpec((NC, blk, D), lambda i: (0, i, 0))],
        out_specs=pl.BlockSpec((blk, D), lambda i: (i, 0)),
        out_shape=jax.ShapeDtypeStruct((N, D), jnp.float32),
    )(partials)
    return h


# speedup vs baseline: 10.7081x; 10.7081x over previous
import functools

import jax
import jax.numpy as jnp
from jax import lax
from jax.experimental import pallas as pl
from jax.experimental.pallas import tpu as pltpu
from jax.experimental.pallas import tpu_sc as plsc

N = 10000
D = 128
K = 128
NC = 2
NS = 16
NW = NC * NS
ROWS_PER_TILE = N // NS
ZROWS = 25


def _matvec_body(x_ref, w_ref, o_ref):
    o_ref[...] = jax.lax.dot_general(
        w_ref[...], x_ref[...], (((1,), (1,)), ((), ())),
        preferred_element_type=jnp.float32)


def _combine_body(p_ref, o_ref):
    o_ref[...] = p_ref[0] + p_ref[1]


def _sc_body(ept, x_hbm, srcp, dstp, ewp, a2_hbm, out_hbm,
             h_sh, asrc_v, adst_v, src_v, dst_v, ew_v, coeff_v, xrows, zbuf,
             sem):
    c = lax.axis_index("c")
    s = lax.axis_index("s")
    wid = s * NC + c

    pltpu.sync_copy(a2_hbm.at[0], asrc_v)
    pltpu.sync_copy(a2_hbm.at[1], adst_v)

    @pl.loop(0, ZROWS)
    def _zero(r):
        for t in range(D // 16):
            zbuf[r, pl.ds(t * 16, 16)] = jnp.zeros((16,), jnp.float32)

    for z in range(ROWS_PER_TILE // ZROWS):
        pltpu.sync_copy(zbuf, h_sh.at[pl.ds(s * ROWS_PER_TILE + z * ZROWS, ZROWS)])
    plsc.subcore_barrier()

    @pl.loop(0, ept // K)
    def _chunk(i):
        base = wid * ept + i * K
        pltpu.sync_copy(srcp.at[pl.ds(base, K)], src_v)
        pltpu.sync_copy(dstp.at[pl.ds(base, K)], dst_v)
        pltpu.sync_copy(ewp.at[pl.ds(base, K)], ew_v)

        gather = pltpu.async_copy(x_hbm.at[src_v], xrows, sem)

        for t in range(K // 16):
            sl = pl.ds(t * 16, 16)
            a_s = plsc.load_gather(asrc_v, [src_v[sl]])
            a_d = plsc.load_gather(adst_v, [dst_v[sl]])
            z = a_s + a_d
            th = 1.0 - 2.0 / (jnp.exp(z + z) + 1.0)
            coeff_v[sl] = th * ew_v[sl]

        gather.wait()

        @pl.loop(0, K)
        def _scale(j):
            cvec = plsc.load_gather(coeff_v, [jnp.full((16,), j, jnp.int32)])
            for t in range(D // 16):
                sl = pl.ds(t * 16, 16)
                xrows[j, sl] = xrows[j, sl] * cvec

        pltpu.sync_copy(xrows, h_sh.at[dst_v], add=True)

    plsc.subcore_barrier()

    pltpu.sync_copy(h_sh.at[pl.ds(s * ROWS_PER_TILE, ROWS_PER_TILE)],
                    out_hbm.at[c, pl.ds(s * ROWS_PER_TILE, ROWS_PER_TILE)])


def kernel(x, edge_index, ew, w_src, w_dst):
    E = ew.shape[0]
    src = edge_index[0].astype(jnp.int32)
    dst = edge_index[1].astype(jnp.int32)

    ept = -(-E // (NW * K)) * K
    e_pad = ept * NW
    pad = e_pad - E
    if pad:
        src = jnp.concatenate([src, jnp.zeros((pad,), jnp.int32)])
        dst = jnp.concatenate([dst, jnp.zeros((pad,), jnp.int32)])
        ew = jnp.concatenate([ew, jnp.zeros((pad,), jnp.float32)])

    wb = jnp.stack([w_src, w_dst])
    blk = 1000
    a2 = pl.pallas_call(
        _matvec_body,
        out_shape=jax.ShapeDtypeStruct((2, N), jnp.float32),
    )(x, wb)

    sc = pl.kernel(
        functools.partial(_sc_body, ept),
        out_type=jax.ShapeDtypeStruct((NC, N, D), jnp.float32),
        mesh=plsc.VectorSubcoreMesh(core_axis_name="c", subcore_axis_name="s"),
        compiler_params=pltpu.CompilerParams(
            use_tc_tiling_on_sc=False, needs_layout_passes=False),
        scratch_types=[
            pltpu.VMEM_SHARED((N, D), jnp.float32),
            pltpu.VMEM((N,), jnp.float32),
            pltpu.VMEM((N,), jnp.float32),
            pltpu.VMEM((K,), jnp.int32),
            pltpu.VMEM((K,), jnp.int32),
            pltpu.VMEM((K,), jnp.float32),
            pltpu.VMEM((K,), jnp.float32),
            pltpu.VMEM((K, D), jnp.float32),
            pltpu.VMEM((ZROWS, D), jnp.float32),
            pltpu.SemaphoreType.DMA,
        ],
    )
    partials = sc(x, src, dst, ew, a2)

    h = pl.pallas_call(
        _combine_body,
        grid=(N // blk,),
        in_specs=[pl.BlockSpec((NC, blk, D), lambda i: (0, i, 0))],
        out_specs=pl.BlockSpec((blk, D), lambda i: (i, 0)),
        out_shape=jax.ShapeDtypeStruct((N, D), jnp.float32),
    )(partials)
    return h


# speedup vs baseline: 18.5377x; 1.7312x over previous
import functools

import jax
import jax.numpy as jnp
from jax import lax
from jax.experimental import pallas as pl
from jax.experimental.pallas import tpu as pltpu
from jax.experimental.pallas import tpu_sc as plsc

N = 10000
D = 128
K = 96
NC = 2
NS = 16
NW = NC * NS
ROWS_PER_TILE = N // NS
ZROWS = 25


def _matvec_body(x_ref, w_ref, o_ref):
    o_ref[...] = jax.lax.dot_general(
        w_ref[...], x_ref[...], (((1,), (1,)), ((), ())),
        preferred_element_type=jnp.float32)


def _combine_body(p_ref, o_ref):
    o_ref[...] = p_ref[0] + p_ref[1]


def _sc_body(nch, x_hbm, edata, dstp, a2_hbm, out_hbm,
             h_sh, asrc_v, adst_v, eb, dstb, coeff_v, xr, zbuf,
             sem_e, sem_g, sem_s):
    c = lax.axis_index("c")
    s = lax.axis_index("s")
    wid = s * NC + c
    cbase = wid * nch
    ebase = wid * nch * K

    pltpu.sync_copy(a2_hbm.at[0], asrc_v)
    pltpu.sync_copy(a2_hbm.at[1], adst_v)

    @pl.loop(0, ZROWS)
    def _zero(r):
        for t in range(D // 16):
            zbuf[r, pl.ds(t * 16, 16)] = jnp.zeros((16,), jnp.float32)

    for z in range(ROWS_PER_TILE // ZROWS):
        pltpu.sync_copy(zbuf, h_sh.at[pl.ds(s * ROWS_PER_TILE + z * ZROWS, ZROWS)])
    plsc.subcore_barrier()

    def issue_ed(ci, be, bd):
        pltpu.async_copy(edata.at[cbase + ci], eb.at[be], sem_e)
        pltpu.async_copy(dstp.at[pl.ds(ebase + ci * K, K)], dstb.at[bd], sem_e)

    def drain_ed():
        pltpu.make_async_copy(edata.at[cbase], eb.at[0], sem_e).wait()
        pltpu.make_async_copy(dstp.at[pl.ds(ebase, K)], dstb.at[0], sem_e).wait()

    def issue_gather(ci, be, bx):
        pltpu.async_copy(x_hbm.at[eb.at[be, 0]], xr.at[bx], sem_g)

    def drain_gather():
        pltpu.make_async_copy(x_hbm.at[eb.at[0, 0]], xr.at[0], sem_g).wait()

    def drain_scatter():
        pltpu.make_async_copy(xr.at[0], h_sh.at[dstb.at[0]], sem_s).wait()

    pltpu.sync_copy(edata.at[cbase], eb.at[0])
    pltpu.sync_copy(dstp.at[pl.ds(ebase, K)], dstb.at[0])
    issue_ed(jnp.minimum(1, nch - 1), 1 % 3, 1 % 4)
    issue_gather(0, 0, 0)

    @pl.loop(0, nch)
    def _chunk(i):
        b3 = lax.rem(i, 3)
        b4 = lax.rem(i, 4)
        b2 = lax.rem(i, 2)

        issue_ed(jnp.minimum(i + 2, nch - 1), lax.rem(i + 2, 3), lax.rem(i + 2, 4))

        for t in range(K // 16):
            sl = pl.ds(t * 16, 16)
            s16 = eb[b3, 0, sl]
            ew16 = plsc.bitcast(eb[b3, 1, sl], jnp.float32)
            d16 = dstb[b4, sl]
            a_s = plsc.load_gather(asrc_v, [s16])
            a_d = plsc.load_gather(adst_v, [d16])
            z = a_s + a_d
            th = 1.0 - 2.0 / (jnp.exp(z + z) + 1.0)
            coeff_v[sl] = th * ew16

        drain_ed()

        @pl.when(i >= 1)
        def _():
            drain_scatter()

        issue_gather(jnp.minimum(i + 1, nch - 1), lax.rem(i + 1, 3), lax.rem(i + 1, 2))

        drain_gather()

        @pl.loop(0, K)
        def _scale(j):
            cvec = plsc.load_gather(coeff_v, [jnp.full((16,), j, jnp.int32)])
            for t in range(D // 16):
                sl = pl.ds(t * 16, 16)
                xr[b2, j, sl] = xr[b2, j, sl] * cvec

        pltpu.async_copy(xr.at[b2], h_sh.at[dstb.at[b4]], sem_s, add=True)

    drain_scatter()
    drain_gather()
    drain_ed()

    plsc.subcore_barrier()

    pltpu.sync_copy(h_sh.at[pl.ds(s * ROWS_PER_TILE, ROWS_PER_TILE)],
                    out_hbm.at[c, pl.ds(s * ROWS_PER_TILE, ROWS_PER_TILE)])


def kernel(x, edge_index, ew, w_src, w_dst):
    E = ew.shape[0]
    src = edge_index[0].astype(jnp.int32)
    dst = edge_index[1].astype(jnp.int32)

    ept = -(-E // (NW * K)) * K
    e_pad = ept * NW
    pad = e_pad - E
    if pad:
        src = jnp.concatenate([src, jnp.zeros((pad,), jnp.int32)])
        dst = jnp.concatenate([dst, jnp.zeros((pad,), jnp.int32)])
        ew = jnp.concatenate([ew, jnp.zeros((pad,), jnp.float32)])
    nch = ept // K

    ewbits = lax.bitcast_convert_type(ew, jnp.int32)
    edata = jnp.stack([src.reshape(-1, K), ewbits.reshape(-1, K)], axis=1)

    wb = jnp.stack([w_src, w_dst])
    a2 = pl.pallas_call(
        _matvec_body,
        out_shape=jax.ShapeDtypeStruct((2, N), jnp.float32),
    )(x, wb)

    sc = pl.kernel(
        functools.partial(_sc_body, nch),
        out_type=jax.ShapeDtypeStruct((NC, N, D), jnp.float32),
        mesh=plsc.VectorSubcoreMesh(core_axis_name="c", subcore_axis_name="s"),
        compiler_params=pltpu.CompilerParams(
            use_tc_tiling_on_sc=False, needs_layout_passes=False),
        scratch_types=[
            pltpu.VMEM_SHARED((N, D), jnp.float32),
            pltpu.VMEM((N,), jnp.float32),
            pltpu.VMEM((N,), jnp.float32),
            pltpu.VMEM((3, 2, K), jnp.int32),
            pltpu.VMEM((4, K), jnp.int32),
            pltpu.VMEM((K,), jnp.float32),
            pltpu.VMEM((2, K, D), jnp.float32),
            pltpu.VMEM((ZROWS, D), jnp.float32),
            pltpu.SemaphoreType.DMA,
            pltpu.SemaphoreType.DMA,
            pltpu.SemaphoreType.DMA,
        ],
    )
    partials = sc(x, edata, dst, a2)

    blk = 1000
    h = pl.pallas_call(
        _combine_body,
        grid=(N // blk,),
        in_specs=[pl.BlockSpec((NC, blk, D), lambda i: (0, i, 0))],
        out_specs=pl.BlockSpec((blk, D), lambda i: (i, 0)),
        out_shape=jax.ShapeDtypeStruct((N, D), jnp.float32),
    )(partials)
    return h


# speedup vs baseline: 18.8350x; 1.0160x over previous
import functools

import jax
import jax.numpy as jnp
from jax import lax
from jax.experimental import pallas as pl
from jax.experimental.pallas import tpu as pltpu
from jax.experimental.pallas import tpu_sc as plsc

N = 10000
D = 128
K = 112
NC = 2
NS = 16
NW = NC * NS
ROWS_PER_TILE = N // NS


def _matvec_body(x_ref, w_ref, o_ref):
    o_ref[...] = jax.lax.dot_general(
        w_ref[...], x_ref[...], (((1,), (1,)), ((), ())),
        preferred_element_type=jnp.float32)


def _combine_body(p_ref, o_ref):
    o_ref[...] = p_ref[0] + p_ref[1]


def _sc_body(nch, x_hbm, edata, dstp, a2_hbm, out_hbm,
             h_sh, asrc_v, adst_v, eb, dstb, coeff_v, xr,
             sem_e, sem_g, sem_s):
    c = lax.axis_index("c")
    s = lax.axis_index("s")
    wid = s * NC + c
    cbase = wid * nch
    ebase = wid * nch * K

    pltpu.sync_copy(a2_hbm.at[0], asrc_v)
    pltpu.sync_copy(a2_hbm.at[1], adst_v)

    @pl.loop(0, K, unroll=4)
    def _zero(r):
        for t in range(D // 16):
            xr[0, r, pl.ds(t * 16, 16)] = jnp.zeros((16,), jnp.float32)

    nfull = ROWS_PER_TILE // K
    rem = ROWS_PER_TILE - nfull * K
    for z in range(nfull):
        pltpu.sync_copy(xr.at[0], h_sh.at[pl.ds(s * ROWS_PER_TILE + z * K, K)])
    if rem:
        pltpu.sync_copy(xr.at[0, pl.ds(0, rem)],
                        h_sh.at[pl.ds(s * ROWS_PER_TILE + nfull * K, rem)])
    plsc.subcore_barrier()

    def issue_ed(ci, be, bd):
        pltpu.async_copy(edata.at[cbase + ci], eb.at[be], sem_e)
        pltpu.async_copy(dstp.at[pl.ds(ebase + ci * K, K)], dstb.at[bd], sem_e)

    def drain_ed():
        pltpu.make_async_copy(edata.at[cbase], eb.at[0], sem_e).wait()
        pltpu.make_async_copy(dstp.at[pl.ds(ebase, K)], dstb.at[0], sem_e).wait()

    def issue_gather(ci, be, bx):
        pltpu.async_copy(x_hbm.at[eb.at[be, 0]], xr.at[bx], sem_g)

    def drain_gather():
        pltpu.make_async_copy(x_hbm.at[eb.at[0, 0]], xr.at[0], sem_g).wait()

    def drain_scatter():
        pltpu.make_async_copy(xr.at[0], h_sh.at[dstb.at[0]], sem_s).wait()

    pltpu.sync_copy(edata.at[cbase], eb.at[0])
    pltpu.sync_copy(dstp.at[pl.ds(ebase, K)], dstb.at[0])
    issue_ed(jnp.minimum(1, nch - 1), 1 % 3, 1 % 4)
    issue_gather(0, 0, 0)

    @pl.loop(0, nch)
    def _chunk(i):
        b3 = lax.rem(i, 3)
        b4 = lax.rem(i, 4)
        b2 = lax.rem(i, 2)

        issue_ed(jnp.minimum(i + 2, nch - 1), lax.rem(i + 2, 3), lax.rem(i + 2, 4))

        for t in range(K // 16):
            sl = pl.ds(t * 16, 16)
            s16 = eb[b3, 0, sl]
            ew16 = plsc.bitcast(eb[b3, 1, sl], jnp.float32)
            d16 = dstb[b4, sl]
            a_s = plsc.load_gather(asrc_v, [s16])
            a_d = plsc.load_gather(adst_v, [d16])
            z = a_s + a_d
            th = 1.0 - 2.0 / (jnp.exp(z + z) + 1.0)
            coeff_v[sl] = th * ew16

        drain_ed()

        @pl.when(i >= 1)
        def _():
            drain_scatter()

        issue_gather(jnp.minimum(i + 1, nch - 1), lax.rem(i + 1, 3), lax.rem(i + 1, 2))

        drain_gather()

        @pl.loop(0, K, unroll=4)
        def _scale(j):
            cvec = plsc.load_gather(coeff_v, [jnp.full((16,), j, jnp.int32)])
            for t in range(D // 16):
                sl = pl.ds(t * 16, 16)
                xr[b2, j, sl] = xr[b2, j, sl] * cvec

        pltpu.async_copy(xr.at[b2], h_sh.at[dstb.at[b4]], sem_s, add=True)

    drain_scatter()
    drain_gather()
    drain_ed()

    plsc.subcore_barrier()

    pltpu.sync_copy(h_sh.at[pl.ds(s * ROWS_PER_TILE, ROWS_PER_TILE)],
                    out_hbm.at[c, pl.ds(s * ROWS_PER_TILE, ROWS_PER_TILE)])


def kernel(x, edge_index, ew, w_src, w_dst):
    E = ew.shape[0]
    src = edge_index[0].astype(jnp.int32)
    dst = edge_index[1].astype(jnp.int32)

    ept = -(-E // (NW * K)) * K
    e_pad = ept * NW
    pad = e_pad - E
    if pad:
        src = jnp.concatenate([src, jnp.zeros((pad,), jnp.int32)])
        dst = jnp.concatenate([dst, jnp.zeros((pad,), jnp.int32)])
        ew = jnp.concatenate([ew, jnp.zeros((pad,), jnp.float32)])
    nch = ept // K

    ewbits = lax.bitcast_convert_type(ew, jnp.int32)
    edata = jnp.stack([src.reshape(-1, K), ewbits.reshape(-1, K)], axis=1)

    wb = jnp.stack([w_src, w_dst])
    a2 = pl.pallas_call(
        _matvec_body,
        out_shape=jax.ShapeDtypeStruct((2, N), jnp.float32),
    )(x, wb)

    sc = pl.kernel(
        functools.partial(_sc_body, nch),
        out_type=jax.ShapeDtypeStruct((NC, N, D), jnp.float32),
        mesh=plsc.VectorSubcoreMesh(core_axis_name="c", subcore_axis_name="s"),
        compiler_params=pltpu.CompilerParams(
            use_tc_tiling_on_sc=False, needs_layout_passes=False),
        scratch_types=[
            pltpu.VMEM_SHARED((N, D), jnp.float32),
            pltpu.VMEM((N,), jnp.float32),
            pltpu.VMEM((N,), jnp.float32),
            pltpu.VMEM((3, 2, K), jnp.int32),
            pltpu.VMEM((4, K), jnp.int32),
            pltpu.VMEM((K,), jnp.float32),
            pltpu.VMEM((2, K, D), jnp.float32),
            pltpu.SemaphoreType.DMA,
            pltpu.SemaphoreType.DMA,
            pltpu.SemaphoreType.DMA,
        ],
    )
    partials = sc(x, edata, dst, a2)

    blk = 1000
    h = pl.pallas_call(
        _combine_body,
        grid=(N // blk,),
        in_specs=[pl.BlockSpec((NC, blk, D), lambda i: (0, i, 0))],
        out_specs=pl.BlockSpec((blk, D), lambda i: (i, 0)),
        out_shape=jax.ShapeDtypeStruct((N, D), jnp.float32),
    )(partials)
    return h
